# baseline (speedup 1.0000x reference)
import functools

import jax
import jax.numpy as jnp
from jax.experimental import pallas as pl
from jax.experimental.pallas import tpu as pltpu

_NSEG = 512
_SEGCHUNK = 128
_NCHUNK = _NSEG // _SEGCHUNK
_ACC_ROWS = _NSEG + _SEGCHUNK


def _body(batch_ref, x_ref, w1_ref, b1_ref, w2_ref, b2_ref, out_ref,
          acc_ref, den_ref, *, nblocks, bn):
    blk = pl.program_id(0)

    @pl.when(blk == 0)
    def _init():
        acc_ref[...] = jnp.zeros_like(acc_ref)
        den_ref[...] = jnp.zeros_like(den_ref)

    xb16 = x_ref[...].astype(jnp.bfloat16)
    h = jnp.tanh(
        jax.lax.dot_general(xb16, w1_ref[...], (((1,), (0,)), ((), ())),
                            preferred_element_type=jnp.float32)
        + b1_ref[...])
    lrow = jax.lax.dot_general(w2_ref[...], h.astype(jnp.bfloat16),
                               (((0,), (1,)), ((), ())),
                               preferred_element_type=jnp.float32)
    erow = jnp.exp(lrow + b2_ref[0, 0]).astype(jnp.bfloat16)

    brow = batch_ref[0]
    bmin = batch_ref[0, 0, 0]
    bmax = batch_ref[0, 0, bn - 1]
    base = (bmin // 8) * 8
    ones8 = jnp.ones((bn, 8), jnp.bfloat16)
    subl = jax.lax.broadcasted_iota(jnp.int16, (_SEGCHUNK, bn), 0)

    def _scatter(anchor, sl):
        rel = (brow - anchor).astype(jnp.int16)
        owt = jnp.where(rel == subl, erow, jnp.bfloat16(0))
        num = jax.lax.dot_general(owt, xb16, (((1,), (0,)), ((), ())),
                                  preferred_element_type=jnp.float32)
        dcol = jax.lax.dot_general(owt, ones8, (((1,), (0,)), ((), ())),
                                   preferred_element_type=jnp.float32)
        acc_ref[sl, :] = acc_ref[sl, :] + num
        den_ref[sl, :] = den_ref[sl, :] + dcol[:, 0:1]

    @pl.when(bmax - base < _SEGCHUNK)
    def _fast():
        _scatter(base, pl.ds(base, _SEGCHUNK))

    @pl.when(bmax - base >= _SEGCHUNK)
    def _slow():
        for c in range(_NCHUNK):
            @pl.when((bmin < (c + 1) * _SEGCHUNK) & (bmax >= c * _SEGCHUNK))
            def _chunk(c=c):
                _scatter(c * _SEGCHUNK, pl.ds(c * _SEGCHUNK, _SEGCHUNK))

    @pl.when(blk == nblocks - 1)
    def _finish():
        out_ref[...] = acc_ref[0:_NSEG, :] / (den_ref[0:_NSEG, :] + 1e-16)


def kernel(x, batch, W1, b1, W2, b2):
    n, d = x.shape
    bn = 10000
    nblocks = pl.cdiv(n, bn)

    batch3d = batch.reshape(nblocks, 1, bn)
    b1r = b1.reshape(1, d)
    w2col = W2.astype(jnp.bfloat16)
    b2r = b2.reshape(1, 1)
    w1_16 = W1.astype(jnp.bfloat16)

    out = pl.pallas_call(
        functools.partial(_body, nblocks=nblocks, bn=bn),
        grid=(nblocks,),
        in_specs=[
            pl.BlockSpec((1, 1, bn), lambda i: (i, 0, 0)),
            pl.BlockSpec((bn, d), lambda i: (i, 0)),
            pl.BlockSpec((d, d), lambda i: (0, 0)),
            pl.BlockSpec((1, d), lambda i: (0, 0)),
            pl.BlockSpec((d, 1), lambda i: (0, 0)),
            pl.BlockSpec((1, 1), lambda i: (0, 0)),
        ],
        out_specs=pl.BlockSpec((_NSEG, d), lambda i: (0, 0)),
        out_shape=jax.ShapeDtypeStruct((_NSEG, d), jnp.float32),
        scratch_shapes=[
            pltpu.VMEM((_ACC_ROWS, d), jnp.float32),
            pltpu.VMEM((_ACC_ROWS, 1), jnp.float32),
        ],
        compiler_params=pltpu.CompilerParams(
            dimension_semantics=("arbitrary",),
        ),
    )(batch3d, x, w1_16, b1r, w2col, b2r)
    return out
